# baseline (speedup 1.0000x reference)
import functools

import jax
import jax.numpy as jnp
from jax import lax
from jax.experimental import pallas as pl
from jax.experimental.pallas import tpu as pltpu
from jax.experimental.pallas import tpu_sc as plsc

VOC = 100000
D = 64
DP = 128
N_MEM = 200
T_Q = 20
T_M = 50
N_HOPS = 3
L = 16
NC = 2
NS = 16
SLOTS_PER_W = 8
N_STORY_W = N_MEM // SLOTS_PER_W

WBLK = 5000
N_WSTEP = VOC // WBLK

VBLK = 5000
N_VSTEP = VOC // VBLK


def _widen_body(wa_ref, wc_ref, wa2_ref, wc2_ref):
    wa = wa_ref[...]
    wc = wc_ref[...]
    wa2_ref[...] = jnp.concatenate([wa, wa], axis=1)
    wc2_ref[...] = jnp.concatenate([wc, wc], axis=1)


def _widen(Wa, Wc):
    return pl.pallas_call(
        _widen_body,
        grid=(N_WSTEP,),
        out_shape=[
            jax.ShapeDtypeStruct((VOC, DP), jnp.float32),
            jax.ShapeDtypeStruct((VOC, DP), jnp.float32),
        ],
        in_specs=[
            pl.BlockSpec((WBLK, D), lambda i: (i, 0)),
            pl.BlockSpec((WBLK, D), lambda i: (i, 0)),
        ],
        out_specs=[
            pl.BlockSpec((WBLK, DP), lambda i: (i, 0)),
            pl.BlockSpec((WBLK, DP), lambda i: (i, 0)),
        ],
    )(Wa, Wc)


def _slot_sum(rows_ref, j, out_ref, n_rows):
    zero = jnp.zeros((L,), jnp.float32)

    def body(r, accs):
        return tuple(
            accs[c] + rows_ref[j, r, pl.ds(c * L, L)]
            for c in range(D // L)
        )

    accs = lax.fori_loop(0, n_rows, body, (zero,) * (D // L))
    for c in range(D // L):
        out_ref[j, pl.ds(c * L, L)] = accs[c]


def _sc_body(story_hbm, wa_hbm, wc_hbm,
             mem_in_hbm, mem_out_hbm,
             idx_v, rows_a, rows_c, acc_in, acc_out, sem):
    wid = lax.axis_index("c") * NS + lax.axis_index("s")

    @pl.when(wid < N_STORY_W)
    def _story_work():
        base = wid * SLOTS_PER_W
        pltpu.sync_copy(story_hbm.at[pl.ds(base, SLOTS_PER_W)], idx_v)
        copies = []
        for j in range(SLOTS_PER_W):
            copies.append(pltpu.async_copy(
                wa_hbm.at[idx_v.at[j, pl.ds(0, T_M)]], rows_a.at[j], sem))
            copies.append(pltpu.async_copy(
                wc_hbm.at[idx_v.at[j, pl.ds(0, T_M)]], rows_c.at[j], sem))
        for cp in copies:
            cp.wait()
        for j in range(SLOTS_PER_W):
            _slot_sum(rows_a, j, acc_in, T_M)
            _slot_sum(rows_c, j, acc_out, T_M)
        pltpu.sync_copy(acc_in, mem_in_hbm.at[pl.ds(base, SLOTS_PER_W)])
        pltpu.sync_copy(acc_out, mem_out_hbm.at[pl.ds(base, SLOTS_PER_W)])


_sc_gather_sums = functools.partial(
    pl.kernel,
    out_type=[
        jax.ShapeDtypeStruct((N_MEM, DP), jnp.float32),
        jax.ShapeDtypeStruct((N_MEM, DP), jnp.float32),
    ],
    mesh=plsc.VectorSubcoreMesh(core_axis_name="c", subcore_axis_name="s"),
    compiler_params=pltpu.CompilerParams(use_tc_tiling_on_sc=True),
    scratch_types=[
        pltpu.VMEM((SLOTS_PER_W, DP), jnp.int32),
        pltpu.VMEM((SLOTS_PER_W, T_M, DP), jnp.float32),
        pltpu.VMEM((SLOTS_PER_W, T_M, DP), jnp.float32),
        pltpu.VMEM((SLOTS_PER_W, DP), jnp.float32),
        pltpu.VMEM((SLOTS_PER_W, DP), jnp.float32),
        pltpu.SemaphoreType.DMA,
    ],
)(_sc_body)


def _tc_body(query_smem, mem_in_ref, mem_out_ref, ta_ref, tc_ref,
             hw_ref, hb_ref, wo_ref, wb_hbm, out_ref, lse_ref,
             qblk_ref, u_ref, m_ref, s_ref, sem):
    step = pl.program_id(0)

    @pl.when(step == 0)
    def _hops():
        copies = []
        for t in range(T_Q):
            q = query_smem[0, t]
            start = pl.multiple_of((q // 8) * 8, 8)
            copies.append(pltpu.make_async_copy(
                wb_hbm.at[pl.ds(start, 8)], qblk_ref.at[pl.ds(t * 8, 8)],
                sem))
        for cp in copies:
            cp.start()
        for cp in copies:
            cp.wait()
        rid = lax.broadcasted_iota(jnp.int32, (T_Q * 8, 1), 0)
        mask = jnp.zeros((T_Q * 8, 1), jnp.float32)
        for t in range(T_Q):
            qmod = lax.rem(query_smem[0, t], 8)
            mask = mask + jnp.where(rid == t * 8 + qmod, 1.0, 0.0)
        u = lax.dot_general(mask, qblk_ref[...], (((0,), (0,)), ((), ())),
                            preferred_element_type=jnp.float32)

        mem_in = mem_in_ref[...][:, :D] + ta_ref[...]
        mem_out = mem_out_ref[...][:, :D] + tc_ref[...]
        hw = hw_ref[...]
        hb = hb_ref[...]
        for _ in range(N_HOPS):
            attn = lax.dot_general(mem_in, u, (((1,), (1,)), ((), ())),
                                   preferred_element_type=jnp.float32)
            attn = attn - jnp.max(attn, axis=0, keepdims=True)
            e = jnp.exp(attn)
            p = e / jnp.sum(e, axis=0, keepdims=True)
            wrow = lax.dot_general(p, mem_out, (((0,), (0,)), ((), ())),
                                   preferred_element_type=jnp.float32)
            u = u + lax.dot_general(wrow, hw, (((1,), (1,)), ((), ())),
                                    preferred_element_type=jnp.float32) + hb
        u_ref[...] = u
        m_ref[0, 0] = -jnp.inf
        s_ref[0, 0] = 0.0

    lb = lax.dot_general(u_ref[...], wo_ref[...], (((1,), (1,)), ((), ())),
                         preferred_element_type=jnp.float32)
    out_ref[...] = lb.reshape(1, 1, VBLK)
    bm = jnp.max(lb)
    m_old = m_ref[0, 0]
    m_new = jnp.maximum(m_old, bm)
    s_ref[0, 0] = (s_ref[0, 0] * jnp.exp(m_old - m_new)
                   + jnp.sum(jnp.exp(lb - m_new)))
    m_ref[0, 0] = m_new

    @pl.when(step == N_VSTEP - 1)
    def _finish():
        lse_ref[0, 0] = m_ref[0, 0] + jnp.log(s_ref[0, 0])


def _sub_lse_body(logits_ref, lse_smem, out_ref):
    out_ref[...] = logits_ref[...] - lse_smem[0, 0]


def _tc_finish(query, mem_in, mem_out, TA, TC_pos, H_w, H_b_row, weight_out,
               Wb):
    raw, lse = pl.pallas_call(
        _tc_body,
        grid=(N_VSTEP,),
        out_shape=[
            jax.ShapeDtypeStruct((N_VSTEP, 1, VBLK), jnp.float32),
            jax.ShapeDtypeStruct((1, 1), jnp.float32),
        ],
        in_specs=[
            pl.BlockSpec(memory_space=pltpu.SMEM),
            pl.BlockSpec((N_MEM, DP), lambda i: (0, 0)),
            pl.BlockSpec((N_MEM, DP), lambda i: (0, 0)),
            pl.BlockSpec((N_MEM, D), lambda i: (0, 0)),
            pl.BlockSpec((N_MEM, D), lambda i: (0, 0)),
            pl.BlockSpec((D, D), lambda i: (0, 0)),
            pl.BlockSpec((1, D), lambda i: (0, 0)),
            pl.BlockSpec((VBLK, D), lambda i: (i, 0)),
            pl.BlockSpec(memory_space=pl.ANY),
        ],
        out_specs=[
            pl.BlockSpec((1, 1, VBLK), lambda i: (i, 0, 0)),
            pl.BlockSpec(memory_space=pltpu.SMEM),
        ],
        scratch_shapes=[
            pltpu.VMEM((T_Q * 8, D), jnp.float32),
            pltpu.VMEM((1, D), jnp.float32),
            pltpu.SMEM((1, 1), jnp.float32),
            pltpu.SMEM((1, 1), jnp.float32),
            pltpu.SemaphoreType.DMA,
        ],
    )(query, mem_in, mem_out, TA, TC_pos, H_w, H_b_row, weight_out, Wb)
    out = pl.pallas_call(
        _sub_lse_body,
        out_shape=jax.ShapeDtypeStruct((N_VSTEP, 1, VBLK), jnp.float32),
        in_specs=[
            pl.BlockSpec(memory_space=pltpu.VMEM),
            pl.BlockSpec(memory_space=pltpu.SMEM),
        ],
    )(raw, lse)
    return out.reshape(1, VOC)


def kernel(query, story, Wa, Wc, Wb, weight_out, H_w, H_b, TA, TC):
    st = story.astype(jnp.int32)
    st = jnp.pad(st, ((0, 0), (0, DP - T_M)))
    q = query.astype(jnp.int32)
    wa2, wc2 = _widen(Wa, Wc)
    mem_in, mem_out = _sc_gather_sums(st, wa2, wc2)
    return _tc_finish(q, mem_in, mem_out, TA, TC, H_w,
                      H_b.reshape(1, D), weight_out, Wb)


# speedup vs baseline: 1.0489x; 1.0489x over previous
import functools

import jax
import jax.numpy as jnp
from jax import lax
from jax.experimental import pallas as pl
from jax.experimental.pallas import tpu as pltpu
from jax.experimental.pallas import tpu_sc as plsc

VOC = 100000
D = 64
DP = 128
VOC2 = VOC // 2
N_MEM = 200
T_Q = 20
T_M = 50
T_MP = 64
N_HOPS = 3
L = 16
NC = 2
NS = 16
SLOTS_PER_W = 8
N_STORY_W = N_MEM // SLOTS_PER_W

PBLK = 4000
N_PSTEP = VOC // PBLK

VBLK = 5000
N_VSTEP = VOC // VBLK


def _pack_body(wa_ref, wc_ref, w2_ref):
    w2_ref[...] = jnp.concatenate([wa_ref[...], wc_ref[...]], axis=1)


def _pack(Wa, Wc):
    return pl.pallas_call(
        _pack_body,
        grid=(N_PSTEP,),
        out_shape=jax.ShapeDtypeStruct((VOC, DP), jnp.float32),
        in_specs=[
            pl.BlockSpec((PBLK, D), lambda i: (i, 0)),
            pl.BlockSpec((PBLK, D), lambda i: (i, 0)),
        ],
        out_specs=pl.BlockSpec((PBLK, DP), lambda i: (i, 0)),
    )(Wa, Wc)


def _slot_sum(rows_ref, j, out_ref, out_row, lane_base, n_rows):
    zero = jnp.zeros((L,), jnp.float32)

    def body(r, accs):
        return tuple(
            accs[c] + rows_ref[j, r, pl.ds(lane_base + c * L, L)]
            for c in range(D // L)
        )

    accs = lax.fori_loop(0, n_rows, body, (zero,) * (D // L))
    for c in range(D // L):
        out_ref[out_row, pl.ds(c * L, L)] = accs[c]


def _sc_body(story_hbm, w2_hbm,
             mem_in_hbm, mem_out_hbm,
             idx_v, rows, acc_in, acc_out, sem):
    wid = lax.axis_index("c") * NS + lax.axis_index("s")

    @pl.when(wid < N_STORY_W)
    def _story_work():
        base = wid * SLOTS_PER_W
        pltpu.sync_copy(story_hbm.at[pl.ds(base, SLOTS_PER_W)], idx_v)
        copies = []
        for j in range(SLOTS_PER_W):
            copies.append(pltpu.async_copy(
                w2_hbm.at[idx_v.at[j]], rows.at[j], sem))
        for cp in copies:
            cp.wait()
        for j in range(SLOTS_PER_W):
            _slot_sum(rows, j, acc_in, j, 0, T_M)
            _slot_sum(rows, j, acc_out, j, D, T_M)
        pltpu.sync_copy(acc_in, mem_in_hbm.at[pl.ds(base, SLOTS_PER_W)])
        pltpu.sync_copy(acc_out, mem_out_hbm.at[pl.ds(base, SLOTS_PER_W)])


_sc_gather_sums = functools.partial(
    pl.kernel,
    out_type=[
        jax.ShapeDtypeStruct((N_MEM, D), jnp.float32),
        jax.ShapeDtypeStruct((N_MEM, D), jnp.float32),
    ],
    mesh=plsc.VectorSubcoreMesh(core_axis_name="c", subcore_axis_name="s"),
    compiler_params=pltpu.CompilerParams(use_tc_tiling_on_sc=False),
    scratch_types=[
        pltpu.VMEM((SLOTS_PER_W, T_M), jnp.int32),
        pltpu.VMEM((SLOTS_PER_W, T_M, DP), jnp.float32),
        pltpu.VMEM((SLOTS_PER_W, D), jnp.float32),
        pltpu.VMEM((SLOTS_PER_W, D), jnp.float32),
        pltpu.SemaphoreType.DMA,
    ],
)(_sc_body)


def _tc_body(query_smem, mem_in_ref, mem_out_ref, ta_ref, tc_ref,
             hw_ref, hb_ref, wo_ref, wb_hbm, out_ref, lse_ref,
             qblk_ref, u_ref, m_ref, s_ref, sem):
    step = pl.program_id(0)

    @pl.when(step == 0)
    def _hops():
        copies = []
        for t in range(T_Q):
            q = query_smem[0, t]
            start = pl.multiple_of((q // 8) * 8, 8)
            copies.append(pltpu.make_async_copy(
                wb_hbm.at[pl.ds(start, 8)], qblk_ref.at[pl.ds(t * 8, 8)],
                sem))
        for cp in copies:
            cp.start()
        for cp in copies:
            cp.wait()
        rid = lax.broadcasted_iota(jnp.int32, (T_Q * 8, 1), 0)
        mask = jnp.zeros((T_Q * 8, 1), jnp.float32)
        for t in range(T_Q):
            qmod = lax.rem(query_smem[0, t], 8)
            mask = mask + jnp.where(rid == t * 8 + qmod, 1.0, 0.0)
        u = lax.dot_general(mask, qblk_ref[...], (((0,), (0,)), ((), ())),
                            preferred_element_type=jnp.float32)

        mem_in = mem_in_ref[...] + ta_ref[...]
        mem_out = mem_out_ref[...] + tc_ref[...]
        hw = hw_ref[...]
        hb = hb_ref[...]
        for _ in range(N_HOPS):
            attn = lax.dot_general(mem_in, u, (((1,), (1,)), ((), ())),
                                   preferred_element_type=jnp.float32)
            attn = attn - jnp.max(attn, axis=0, keepdims=True)
            e = jnp.exp(attn)
            p = e / jnp.sum(e, axis=0, keepdims=True)
            wrow = lax.dot_general(p, mem_out, (((0,), (0,)), ((), ())),
                                   preferred_element_type=jnp.float32)
            u = u + lax.dot_general(wrow, hw, (((1,), (1,)), ((), ())),
                                    preferred_element_type=jnp.float32) + hb
        u_ref[...] = u
        m_ref[0, 0] = -jnp.inf
        s_ref[0, 0] = 0.0

    lb = lax.dot_general(u_ref[...], wo_ref[...], (((1,), (1,)), ((), ())),
                         preferred_element_type=jnp.float32)
    out_ref[...] = lb.reshape(1, 1, VBLK)
    bm = jnp.max(lb)
    m_old = m_ref[0, 0]
    m_new = jnp.maximum(m_old, bm)
    s_ref[0, 0] = (s_ref[0, 0] * jnp.exp(m_old - m_new)
                   + jnp.sum(jnp.exp(lb - m_new)))
    m_ref[0, 0] = m_new

    @pl.when(step == N_VSTEP - 1)
    def _finish():
        lse_ref[0, 0] = m_ref[0, 0] + jnp.log(s_ref[0, 0])


def _sub_lse_body(logits_ref, lse_smem, out_ref):
    out_ref[...] = logits_ref[...] - lse_smem[0, 0]


def _tc_finish(query, mem_in, mem_out, TA, TC_pos, H_w, H_b_row, weight_out,
               Wb):
    raw, lse = pl.pallas_call(
        _tc_body,
        grid=(N_VSTEP,),
        out_shape=[
            jax.ShapeDtypeStruct((N_VSTEP, 1, VBLK), jnp.float32),
            jax.ShapeDtypeStruct((1, 1), jnp.float32),
        ],
        in_specs=[
            pl.BlockSpec(memory_space=pltpu.SMEM),
            pl.BlockSpec((N_MEM, D), lambda i: (0, 0)),
            pl.BlockSpec((N_MEM, D), lambda i: (0, 0)),
            pl.BlockSpec((N_MEM, D), lambda i: (0, 0)),
            pl.BlockSpec((N_MEM, D), lambda i: (0, 0)),
            pl.BlockSpec((D, D), lambda i: (0, 0)),
            pl.BlockSpec((1, D), lambda i: (0, 0)),
            pl.BlockSpec((VBLK, D), lambda i: (i, 0)),
            pl.BlockSpec(memory_space=pl.ANY),
        ],
        out_specs=[
            pl.BlockSpec((1, 1, VBLK), lambda i: (i, 0, 0)),
            pl.BlockSpec(memory_space=pltpu.SMEM),
        ],
        scratch_shapes=[
            pltpu.VMEM((T_Q * 8, D), jnp.float32),
            pltpu.VMEM((1, D), jnp.float32),
            pltpu.SMEM((1, 1), jnp.float32),
            pltpu.SMEM((1, 1), jnp.float32),
            pltpu.SemaphoreType.DMA,
        ],
    )(query, mem_in, mem_out, TA, TC_pos, H_w, H_b_row, weight_out, Wb)
    out = pl.pallas_call(
        _sub_lse_body,
        out_shape=jax.ShapeDtypeStruct((N_VSTEP, 1, VBLK), jnp.float32),
        in_specs=[
            pl.BlockSpec(memory_space=pltpu.VMEM),
            pl.BlockSpec(memory_space=pltpu.SMEM),
        ],
    )(raw, lse)
    return out.reshape(1, VOC)


def kernel(query, story, Wa, Wc, Wb, weight_out, H_w, H_b, TA, TC):
    st = story.astype(jnp.int32)
    q = query.astype(jnp.int32)
    w2 = _pack(Wa, Wc)
    mem_in, mem_out = _sc_gather_sums(st, w2)
    return _tc_finish(q, mem_in, mem_out, TA, TC, H_w,
                      H_b.reshape(1, D), weight_out, Wb)


# speedup vs baseline: 1.1319x; 1.0792x over previous
import functools

import jax
import jax.numpy as jnp
from jax import lax
from jax.experimental import pallas as pl
from jax.experimental.pallas import tpu as pltpu
from jax.experimental.pallas import tpu_sc as plsc

VOC = 100000
D = 64
N_MEM = 200
T_Q = 20
T_M = 50
N_HOPS = 3
L = 16
NC = 2
NS = 16
NW = NC * NS
SLOTS_PER_W = 8
N_STORY_W = N_MEM // SLOTS_PER_W


def _slot_sum(rows_ref, row_base, out_ref, out_row, n_rows):
    zero = jnp.zeros((L,), jnp.float32)

    def body(r, accs):
        return tuple(
            accs[c] + rows_ref[row_base + r, pl.ds(c * L, L)]
            for c in range(D // L)
        )

    accs = lax.fori_loop(0, n_rows, body, (zero,) * (D // L))
    for c in range(D // L):
        out_ref[out_row, pl.ds(c * L, L)] = accs[c]


def _sc_body(story_hbm, wa_hbm, wc_hbm,
             mem_in_hbm, mem_out_hbm,
             idx_v, rows_a, rows_c, acc_in, acc_out, sem):
    wid = lax.axis_index("c") * NS + lax.axis_index("s")

    @pl.when(wid < N_STORY_W)
    def _story_work():
        base = wid * SLOTS_PER_W
        pltpu.sync_copy(story_hbm.at[pl.ds(base, SLOTS_PER_W)], idx_v)
        copies = []
        for j in range(SLOTS_PER_W):
            copies.append(pltpu.async_copy(
                wa_hbm.at[idx_v.at[j]], rows_a.at[pl.ds(j * T_M, T_M)], sem))
            copies.append(pltpu.async_copy(
                wc_hbm.at[idx_v.at[j]], rows_c.at[pl.ds(j * T_M, T_M)], sem))
        for cp in copies:
            cp.wait()
        for j in range(SLOTS_PER_W):
            _slot_sum(rows_a, j * T_M, acc_in, j, T_M)
            _slot_sum(rows_c, j * T_M, acc_out, j, T_M)
        pltpu.sync_copy(acc_in, mem_in_hbm.at[pl.ds(base, SLOTS_PER_W)])
        pltpu.sync_copy(acc_out, mem_out_hbm.at[pl.ds(base, SLOTS_PER_W)])


_sc_gather_sums = functools.partial(
    pl.kernel,
    out_type=[
        jax.ShapeDtypeStruct((N_MEM, D), jnp.float32),
        jax.ShapeDtypeStruct((N_MEM, D), jnp.float32),
    ],
    mesh=plsc.VectorSubcoreMesh(core_axis_name="c", subcore_axis_name="s"),
    compiler_params=pltpu.CompilerParams(use_tc_tiling_on_sc=False),
    scratch_types=[
        pltpu.VMEM((SLOTS_PER_W, T_M), jnp.int32),
        pltpu.VMEM((SLOTS_PER_W * T_M, D), jnp.float32),
        pltpu.VMEM((SLOTS_PER_W * T_M, D), jnp.float32),
        pltpu.VMEM((SLOTS_PER_W, D), jnp.float32),
        pltpu.VMEM((SLOTS_PER_W, D), jnp.float32),
        pltpu.SemaphoreType.DMA,
    ],
)(_sc_body)


def _tc_body(query_smem, mem_in_ref, mem_out_ref, ta_ref, tc_ref,
             hw_ref, hb_ref, wo_ref, wb_hbm, out_ref, qblk_ref, sem):
    copies = []
    for t in range(T_Q):
        q = query_smem[0, t]
        start = pl.multiple_of((q // 8) * 8, 8)
        copies.append(pltpu.make_async_copy(
            wb_hbm.at[pl.ds(start, 8)], qblk_ref.at[pl.ds(t * 8, 8)], sem))
    for cp in copies:
        cp.start()
    for cp in copies:
        cp.wait()
    rid = lax.broadcasted_iota(jnp.int32, (T_Q * 8, 1), 0)
    mask = jnp.zeros((T_Q * 8, 1), jnp.float32)
    for t in range(T_Q):
        qmod = lax.rem(query_smem[0, t], 8)
        mask = mask + jnp.where(rid == t * 8 + qmod, 1.0, 0.0)
    u = lax.dot_general(mask, qblk_ref[...], (((0,), (0,)), ((), ())),
                        preferred_element_type=jnp.float32)

    mem_in = mem_in_ref[...] + ta_ref[...]
    mem_out = mem_out_ref[...] + tc_ref[...]
    hw = hw_ref[...]
    hb = hb_ref[...]
    for _ in range(N_HOPS):
        attn = lax.dot_general(mem_in, u, (((1,), (1,)), ((), ())),
                               preferred_element_type=jnp.float32)
        attn = attn - jnp.max(attn, axis=0, keepdims=True)
        e = jnp.exp(attn)
        p = e / jnp.sum(e, axis=0, keepdims=True)
        wrow = lax.dot_general(p, mem_out, (((0,), (0,)), ((), ())),
                               preferred_element_type=jnp.float32)
        u = u + lax.dot_general(wrow, hw, (((1,), (1,)), ((), ())),
                                preferred_element_type=jnp.float32) + hb
    logits = lax.dot_general(u, wo_ref[...], (((1,), (1,)), ((), ())),
                             preferred_element_type=jnp.float32)
    mx = jnp.max(logits, axis=1, keepdims=True)
    lse = mx + jnp.log(jnp.sum(jnp.exp(logits - mx), axis=1, keepdims=True))
    out_ref[...] = logits - lse


def _tc_finish(query, mem_in, mem_out, TA, TC_pos, H_w, H_b_row, weight_out,
               Wb):
    return pl.pallas_call(
        _tc_body,
        out_shape=jax.ShapeDtypeStruct((1, VOC), jnp.float32),
        in_specs=[
            pl.BlockSpec(memory_space=pltpu.SMEM),
            pl.BlockSpec(memory_space=pltpu.VMEM),
            pl.BlockSpec(memory_space=pltpu.VMEM),
            pl.BlockSpec(memory_space=pltpu.VMEM),
            pl.BlockSpec(memory_space=pltpu.VMEM),
            pl.BlockSpec(memory_space=pltpu.VMEM),
            pl.BlockSpec(memory_space=pltpu.VMEM),
            pl.BlockSpec(memory_space=pltpu.VMEM),
            pl.BlockSpec(memory_space=pl.ANY),
        ],
        scratch_shapes=[
            pltpu.VMEM((T_Q * 8, D), jnp.float32),
            pltpu.SemaphoreType.DMA,
        ],
    )(query, mem_in, mem_out, TA, TC_pos, H_w, H_b_row, weight_out, Wb)


def kernel(query, story, Wa, Wc, Wb, weight_out, H_w, H_b, TA, TC):
    st = story.astype(jnp.int32)
    q = query.astype(jnp.int32)
    mem_in, mem_out = _sc_gather_sums(st, Wa, Wc)
    return _tc_finish(q, mem_in, mem_out, TA, TC, H_w,
                      H_b.reshape(1, D), weight_out, Wb)
